# baseline (speedup 1.0000x reference)
import functools

import jax
import jax.numpy as jnp
from jax import lax
from jax.experimental import pallas as pl
from jax.experimental.pallas import tpu as pltpu
from jax.experimental.pallas import tpu_sc as plsc

_CHUNK = 128
_TBLK = 4096


def _transpose_body(in_ref, out_ref):
    d = in_ref.shape[0]
    r = jax.lax.broadcasted_iota(jnp.int32, (d, d), 0)
    c = jax.lax.broadcasted_iota(jnp.int32, (d, d), 1)
    eye = (r == c).astype(jnp.float32)
    out_ref[...] = jax.lax.dot_general(
        in_ref[...], eye, (((0,), (0,)), ((), ())),
        preferred_element_type=jnp.float32,
    )


@functools.lru_cache(maxsize=None)
def _make_transpose(V, D):
    grid = (V + _TBLK - 1) // _TBLK
    return pl.pallas_call(
        _transpose_body,
        grid=(grid,),
        in_specs=[pl.BlockSpec((D, _TBLK), lambda i: (0, i))],
        out_specs=pl.BlockSpec((_TBLK, D), lambda i: (i, 0)),
        out_shape=jax.ShapeDtypeStruct((V, D), jnp.float32),
    )


@functools.lru_cache(maxsize=None)
def _make_gather(V, D, B):
    info = plsc.get_sparse_core_info()
    NC, NS = info.num_cores, info.num_subcores
    NW = NC * NS
    assert B % (NW * _CHUNK) == 0
    b_per_w = B // NW
    n_chunks = b_per_w // _CHUNK

    mesh = plsc.VectorSubcoreMesh(core_axis_name="c", subcore_axis_name="s")

    @functools.partial(
        pl.kernel,
        mesh=mesh,
        out_type=jax.ShapeDtypeStruct((B, D), jnp.float32),
        scratch_types=[
            pltpu.VMEM((n_chunks, _CHUNK), jnp.int32),
            pltpu.VMEM((b_per_w, D), jnp.float32),
            pltpu.SemaphoreType.DMA,
        ],
        compiler_params=pltpu.CompilerParams(use_tc_tiling_on_sc=False),
    )
    def gather_kernel(table_hbm, idx_hbm, out_hbm, idx_v, rows_v, sem):
        wid = lax.axis_index("s") * NC + lax.axis_index("c")
        pltpu.sync_copy(idx_hbm.at[pl.ds(wid * n_chunks, n_chunks)], idx_v)
        copies = []
        for j in range(n_chunks):
            copies.append(
                pltpu.async_copy(
                    table_hbm.at[idx_v.at[j]],
                    rows_v.at[pl.ds(j * _CHUNK, _CHUNK)],
                    sem,
                )
            )
        for c in copies:
            c.wait()
        pltpu.sync_copy(rows_v, out_hbm.at[pl.ds(wid * b_per_w, b_per_w)])

    return gather_kernel


def kernel(input_ids, embed_weight):
    V, D = embed_weight.shape
    B = input_ids.size
    idx2d = input_ids.reshape(B // _CHUNK, _CHUNK).astype(jnp.int32)
    wt = embed_weight.T
    w_lin = _make_transpose(V, D)(wt)
    out = _make_gather(V, D, B)(w_lin, idx2d)
    return out.reshape(*input_ids.shape, D)


# speedup vs baseline: 1.1464x; 1.1464x over previous
import functools

import jax
import jax.numpy as jnp
from jax import lax
from jax.experimental import pallas as pl
from jax.experimental.pallas import tpu as pltpu
from jax.experimental.pallas import tpu_sc as plsc

_CHUNK = 128
_TBLK = 16384


def _transpose_body(in_ref, out_ref):
    out_ref[...] = in_ref[...].T


@functools.lru_cache(maxsize=None)
def _make_transpose(V, D):
    grid = (V + _TBLK - 1) // _TBLK
    return pl.pallas_call(
        _transpose_body,
        grid=(grid,),
        in_specs=[pl.BlockSpec((D, _TBLK), lambda i: (0, i))],
        out_specs=pl.BlockSpec((_TBLK, D), lambda i: (i, 0)),
        out_shape=jax.ShapeDtypeStruct((V, D), jnp.float32),
    )


@functools.lru_cache(maxsize=None)
def _make_gather(V, D, B):
    info = plsc.get_sparse_core_info()
    NC, NS = info.num_cores, info.num_subcores
    NW = NC * NS
    assert B % (NW * _CHUNK) == 0
    b_per_w = B // NW
    n_chunks = b_per_w // _CHUNK

    mesh = plsc.VectorSubcoreMesh(core_axis_name="c", subcore_axis_name="s")

    @functools.partial(
        pl.kernel,
        mesh=mesh,
        out_type=jax.ShapeDtypeStruct((B, D), jnp.float32),
        scratch_types=[
            pltpu.VMEM((n_chunks, _CHUNK), jnp.int32),
            pltpu.VMEM((b_per_w, D), jnp.float32),
            pltpu.SemaphoreType.DMA,
        ],
        compiler_params=pltpu.CompilerParams(use_tc_tiling_on_sc=False),
    )
    def gather_kernel(table_hbm, idx_hbm, out_hbm, idx_v, rows_v, sem):
        wid = lax.axis_index("s") * NC + lax.axis_index("c")
        pltpu.sync_copy(idx_hbm.at[pl.ds(wid * n_chunks, n_chunks)], idx_v)
        copies = []
        for j in range(n_chunks):
            copies.append(
                pltpu.async_copy(
                    table_hbm.at[idx_v.at[j]],
                    rows_v.at[pl.ds(j * _CHUNK, _CHUNK)],
                    sem,
                )
            )
        for c in copies:
            c.wait()
        pltpu.sync_copy(rows_v, out_hbm.at[pl.ds(wid * b_per_w, b_per_w)])

    return gather_kernel


def kernel(input_ids, embed_weight):
    V, D = embed_weight.shape
    B = input_ids.size
    idx2d = input_ids.reshape(B // _CHUNK, _CHUNK).astype(jnp.int32)
    wt = embed_weight.T
    w_lin = _make_transpose(V, D)(wt)
    out = _make_gather(V, D, B)(w_lin, idx2d)
    return out.reshape(*input_ids.shape, D)
